# Initial kernel scaffold; baseline (speedup 1.0000x reference)
import jax
import jax.numpy as jnp
from jax.experimental import pallas as pl


def kernel(x, adj, W_enc1, b_enc1, W_enc2, b_enc2, W_fc1, b_fc1, W_fc2, b_fc2, W_s1, b_s1, W_s2, b_s2, W_t1, b_t1, W_t2, b_t2):
    raise NotImplementedError("write your pallas kernel here")


# speedup vs baseline: 3.7397x; 3.7397x over previous
import functools

import jax
import jax.numpy as jnp
from jax import lax
from jax.experimental import pallas as pl
from jax.experimental.pallas import tpu as pltpu
from jax.experimental.pallas import tpu_sc as plsc

N = 50000
E = 800000
H = 64

NC = 2
NS = 16
CHUNK = 128
HALF = N // NC
HP = HALF + 88
RPT = HP // NS
NCHUNK_PER_TILE = -(-E // (CHUNK * NS))
NCHUNK = NCHUNK_PER_TILE * NS
EPAD = NCHUNK * CHUNK

CW = 8


def _enc_body(x_ref, w1_ref, b1_ref, w2_ref, b2_ref, h_ref):
    a = jnp.maximum(jnp.dot(x_ref[...], w1_ref[...],
                            preferred_element_type=jnp.float32) + b1_ref[...], 0.0)
    h_ref[...] = jnp.dot(a, w2_ref[...],
                         preferred_element_type=jnp.float32) + b2_ref[...]


def _mlp_body(h_ref, sums_ref, cnt_ref, wf1_ref, bf1_ref, wf2_ref, bf2_ref,
              wh1_ref, bh1_ref, wh2_ref, bh2_ref, out_ref):
    cnt = cnt_ref[...][:, 0:1]
    h = h_ref[...] + sums_ref[...] / jnp.maximum(cnt, 1.0)
    h = jnp.maximum(jnp.dot(h, wf1_ref[...],
                            preferred_element_type=jnp.float32) + bf1_ref[...], 0.0)
    h = jnp.maximum(jnp.dot(h, wf2_ref[...],
                            preferred_element_type=jnp.float32) + bf2_ref[...], 0.0)
    hid = jnp.maximum(jnp.dot(h, wh1_ref[...],
                              preferred_element_type=jnp.float32) + bh1_ref[...], 0.0)
    out_ref[...] = jnp.dot(hid, wh2_ref[...],
                           preferred_element_type=jnp.float32) + bh2_ref[...]


def _sc_aggregate_body(h_hbm, src_hbm, dst_hbm, z64_hbm, zcw_hbm, ones_hbm,
                       sums_out, cnt_out,
                       srcb, dstb, idxb, msgb, onesb, sums_sh, cnt_sh, sem):
    c = lax.axis_index("c")
    s = lax.axis_index("s")
    base = c * HALF

    pltpu.sync_copy(z64_hbm, sums_sh.at[pl.ds(s * RPT, RPT)])
    pltpu.sync_copy(zcw_hbm, cnt_sh.at[pl.ds(s * RPT, RPT)])
    pltpu.sync_copy(ones_hbm, onesb)
    plsc.subcore_barrier()

    def chunk_body(k, carry):
        ci = s * NCHUNK_PER_TILE + k
        pltpu.sync_copy(src_hbm.at[ci], srcb)
        pltpu.sync_copy(dst_hbm.at[ci], dstb)
        pltpu.async_copy(h_hbm.at[dstb], msgb, sem).wait()
        for i in range(CHUNK // 16):
            sl = pl.ds(i * 16, 16)
            rel = srcb[sl] - base
            ok = (rel >= 0) & (rel < HALF)
            idxb[sl] = jnp.where(ok, rel, HALF)
        pltpu.sync_copy(msgb, sums_sh.at[idxb], add=True)
        pltpu.sync_copy(onesb, cnt_sh.at[idxb], add=True)
        return carry

    lax.fori_loop(0, NCHUNK_PER_TILE, chunk_body, 0)
    plsc.subcore_barrier()

    ob = c * HP + s * RPT
    pltpu.sync_copy(sums_sh.at[pl.ds(s * RPT, RPT)], sums_out.at[pl.ds(ob, RPT)])
    pltpu.sync_copy(cnt_sh.at[pl.ds(s * RPT, RPT)], cnt_out.at[pl.ds(ob, RPT)])


_sc_aggregate = functools.partial(
    pl.kernel,
    out_type=[
        jax.ShapeDtypeStruct((NC * HP, H), jnp.float32),
        jax.ShapeDtypeStruct((NC * HP, CW), jnp.float32),
    ],
    mesh=plsc.VectorSubcoreMesh(core_axis_name="c", subcore_axis_name="s"),
    compiler_params=pltpu.CompilerParams(use_tc_tiling_on_sc=False),
    scratch_types=[
        pltpu.VMEM((CHUNK,), jnp.int32),
        pltpu.VMEM((CHUNK,), jnp.int32),
        pltpu.VMEM((CHUNK,), jnp.int32),
        pltpu.VMEM((CHUNK, H), jnp.float32),
        pltpu.VMEM((CHUNK, CW), jnp.float32),
        pltpu.VMEM_SHARED((HP, H), jnp.float32),
        pltpu.VMEM_SHARED((HP, CW), jnp.float32),
        pltpu.SemaphoreType.DMA,
    ],
)(_sc_aggregate_body)


BLK = 2000


def _row_spec(w):
    return pl.BlockSpec((BLK, w), lambda i: (i, 0))


def _full_spec(shape):
    return pl.BlockSpec(shape, lambda i: (0,) * len(shape))


def kernel(x, adj, W_enc1, b_enc1, W_enc2, b_enc2, W_fc1, b_fc1, W_fc2, b_fc2,
           W_s1, b_s1, W_s2, b_s2, W_t1, b_t1, W_t2, b_t2):
    f = x.shape[1]

    h = pl.pallas_call(
        _enc_body,
        grid=(N // BLK,),
        in_specs=[
            _row_spec(f),
            _full_spec((f, H)), _full_spec((1, H)),
            _full_spec((H, H)), _full_spec((1, H)),
        ],
        out_specs=_row_spec(H),
        out_shape=jax.ShapeDtypeStruct((N, H), jnp.float32),
    )(x, W_enc1, b_enc1.reshape(1, H), W_enc2, b_enc2.reshape(1, H))

    pad = EPAD - E
    src = jnp.concatenate([adj[0], jnp.full((pad,), N, jnp.int32)]).reshape(NCHUNK, CHUNK)
    dst = jnp.concatenate([adj[1], jnp.zeros((pad,), jnp.int32)]).reshape(NCHUNK, CHUNK)
    z64 = jnp.zeros((RPT, H), jnp.float32)
    zcw = jnp.zeros((RPT, CW), jnp.float32)
    ones = jnp.ones((CHUNK, CW), jnp.float32)
    sums_p, cnt_p = _sc_aggregate(h, src, dst, z64, zcw, ones)
    sums = jnp.concatenate([sums_p[:HALF], sums_p[HP:HP + HALF]], axis=0)
    cnt = jnp.concatenate([cnt_p[:HALF], cnt_p[HP:HP + HALF]], axis=0)

    wh1 = jnp.concatenate([W_s1, W_t1], axis=1)
    bh1 = jnp.concatenate([b_s1, b_t1]).reshape(1, 64)
    wh2 = jnp.zeros((64, 8), jnp.float32)
    wh2 = wh2.at[:32, 0:1].set(W_s2).at[32:, 1:5].set(W_t2)
    bh2 = jnp.zeros((1, 8), jnp.float32)
    bh2 = bh2.at[0, 0:1].set(b_s2).at[0, 1:5].set(b_t2)

    out8 = pl.pallas_call(
        _mlp_body,
        grid=(N // BLK,),
        in_specs=[
            _row_spec(H), _row_spec(H), _row_spec(CW),
            _full_spec((H, H)), _full_spec((1, H)),
            _full_spec((H, H)), _full_spec((1, H)),
            _full_spec((H, 64)), _full_spec((1, 64)),
            _full_spec((64, 8)), _full_spec((1, 8)),
        ],
        out_specs=_row_spec(8),
        out_shape=jax.ShapeDtypeStruct((N, 8), jnp.float32),
    )(h, sums, cnt, W_fc1, b_fc1.reshape(1, H), W_fc2, b_fc2.reshape(1, H),
      wh1, bh1, wh2, bh2)

    return out8[:, 0], out8[:, 1:5]
